# baseline (speedup 1.0000x reference)
import functools

import jax
import jax.numpy as jnp
from jax import lax
from jax.experimental import pallas as pl
from jax.experimental.pallas import tpu as pltpu
from jax.experimental.pallas import tpu_sc as plsc

N = 10000
E = 320000
D_IN = 128
D_H = 128
C = 64

NC = 2
NS = 16
NW = NC * NS
EPT = E // NW
KD = 40
NCHUNK_D = EPT // KD
EPT_P = 10240
K = 128
NCHUNK = EPT_P // K
NP = 10240
ROWS_PT = NP // NS

_SC_MESH = plsc.VectorSubcoreMesh(core_axis_name="c", subcore_axis_name="s",
                                  num_cores=NC, num_subcores=NS)


@functools.partial(
    pl.kernel,
    out_type=jax.ShapeDtypeStruct((NC, 2, NP), jnp.float32),
    mesh=_SC_MESH,
    scratch_types=[
        pltpu.VMEM((NCHUNK_D, KD), jnp.int32),
        pltpu.VMEM((NCHUNK_D, KD), jnp.int32),
        pltpu.VMEM((KD,), jnp.float32),
        pltpu.SemaphoreType.DMA,
        pltpu.SemaphoreType.DMA,
        pltpu.VMEM_SHARED((NP,), jnp.float32),
        pltpu.VMEM_SHARED((NP,), jnp.float32),
    ],
)
def _sc_degrees(src_hbm, dst_hbm, ones_hbm, zeros_hbm, out_hbm,
                sidx, didx, ones_v, sem_s, sem_d, acc_s, acc_d):
    c = lax.axis_index("c")
    s = lax.axis_index("s")
    wid = c * NS + s
    r0 = s * ROWS_PT
    pltpu.sync_copy(zeros_hbm.at[pl.ds(r0, ROWS_PT)], acc_s.at[pl.ds(r0, ROWS_PT)])
    pltpu.sync_copy(zeros_hbm.at[pl.ds(r0, ROWS_PT)], acc_d.at[pl.ds(r0, ROWS_PT)])
    pltpu.sync_copy(ones_hbm, ones_v)
    pltpu.sync_copy(src_hbm.at[wid], sidx)
    pltpu.sync_copy(dst_hbm.at[wid], didx)
    plsc.subcore_barrier()

    grp = 10

    def body(g, carry):
        c0 = g * grp
        for j in range(grp):
            pltpu.async_copy(ones_v, acc_s.at[sidx.at[c0 + j]], sem_s, add=True)
            pltpu.async_copy(ones_v, acc_d.at[didx.at[c0 + j]], sem_d, add=True)
        for j in range(grp):
            pltpu.make_async_copy(ones_v, acc_s.at[sidx.at[c0 + j]], sem_s).wait()
            pltpu.make_async_copy(ones_v, acc_d.at[didx.at[c0 + j]], sem_d).wait()
        return carry

    lax.fori_loop(0, NCHUNK_D // grp, body, 0)
    plsc.subcore_barrier()
    pltpu.sync_copy(acc_s.at[pl.ds(r0, ROWS_PT)], out_hbm.at[c, 0, pl.ds(r0, ROWS_PT)])
    pltpu.sync_copy(acc_d.at[pl.ds(r0, ROWS_PT)], out_hbm.at[c, 1, pl.ds(r0, ROWS_PT)])


@functools.partial(
    pl.kernel,
    out_type=jax.ShapeDtypeStruct((NC, NP, D_H), jnp.float32),
    mesh=_SC_MESH,
    scratch_types=[
        pltpu.VMEM((K,), jnp.int32),
        pltpu.VMEM((K,), jnp.int32),
        pltpu.VMEM((K,), jnp.int32),
        pltpu.VMEM((K,), jnp.int32),
        pltpu.VMEM((K, D_H), jnp.float32),
        pltpu.VMEM((K, D_H), jnp.float32),
        pltpu.SemaphoreType.DMA,
        pltpu.SemaphoreType.DMA,
        pltpu.SemaphoreType.DMA,
        pltpu.SemaphoreType.DMA,
        pltpu.VMEM_SHARED((NP, D_H), jnp.float32),
    ],
)
def _sc_aggregate(h_hbm, srcf_hbm, dstf_hbm, zeros_hbm, out_hbm,
                  sidxa, sidxb, didxa, didxb, rows_a, rows_b,
                  gsa, gsb, ssa, ssb, acc):
    c = lax.axis_index("c")
    s = lax.axis_index("s")
    wid = c * NS + s
    r0 = s * ROWS_PT
    pltpu.sync_copy(zeros_hbm.at[pl.ds(r0, ROWS_PT)], acc.at[pl.ds(r0, ROWS_PT)])
    plsc.subcore_barrier()

    def do_chunk(i, sidx, didx, rows, gsem, ssem):
        pltpu.sync_copy(srcf_hbm.at[wid, pl.ds(i * K, K)], sidx)
        pltpu.sync_copy(dstf_hbm.at[wid, pl.ds(i * K, K)], didx)
        pltpu.async_copy(h_hbm.at[sidx], rows, gsem).wait()
        pltpu.async_copy(rows, acc.at[didx], ssem, add=True)

    def s_drain(didx, rows, ssem):
        pltpu.make_async_copy(rows, acc.at[didx], ssem).wait()

    do_chunk(0, sidxa, didxa, rows_a, gsa, ssa)
    do_chunk(1, sidxb, didxb, rows_b, gsb, ssb)

    def body(p, carry):
        i = 2 + 2 * p
        s_drain(didxa, rows_a, ssa)
        do_chunk(i, sidxa, didxa, rows_a, gsa, ssa)
        s_drain(didxb, rows_b, ssb)
        do_chunk(i + 1, sidxb, didxb, rows_b, gsb, ssb)
        return carry

    lax.fori_loop(0, NCHUNK // 2 - 1, body, 0)
    s_drain(didxa, rows_a, ssa)
    s_drain(didxb, rows_b, ssb)

    plsc.subcore_barrier()
    pltpu.sync_copy(acc.at[pl.ds(r0, ROWS_PT)], out_hbm.at[c, pl.ds(r0, ROWS_PT)])


BLK = 1000
NBLK = N // BLK


def _norm_cols(d):
    return jnp.where(d > 0, lax.rsqrt(d), 0.0)


def _mm1_body(x_ref, w_ref, o_ref):
    o_ref[...] = jnp.dot(x_ref[...], w_ref[...],
                         preferred_element_type=jnp.float32)


def _scale_body(degs_ref, xw_ref, o_ref):
    ns = _norm_cols(degs_ref[0, 0] + degs_ref[1, 0])
    o_ref[...] = xw_ref[...] * ns


def _mm2_body(degs_ref, p_ref, b1_ref, o_ref):
    ns = _norm_cols(degs_ref[0, 0] + degs_ref[1, 0])
    nd = _norm_cols(degs_ref[0, 1] + degs_ref[1, 1])
    a = p_ref[0] + p_ref[1]
    h = jnp.maximum(a * nd + b1_ref[...], 0.0)
    o_ref[...] = h * ns


def _out_body(degs_ref, p_ref, w_ref, b2_ref, o_ref):
    nd = _norm_cols(degs_ref[0, 1] + degs_ref[1, 1])
    a = (p_ref[0] + p_ref[1]) * nd
    o = jnp.dot(a, w_ref[...], preferred_element_type=jnp.float32) + b2_ref[...]
    m = jnp.max(o, axis=1, keepdims=True)
    e = jnp.exp(o - m)
    o_ref[...] = e / jnp.sum(e, axis=1, keepdims=True)


_DEG_SPEC = pl.BlockSpec((NC, 2, BLK, 1), lambda i: (0, 0, i, 0))


def _tc_mm1(x, w1):
    return pl.pallas_call(
        _mm1_body,
        grid=(NBLK,),
        in_specs=[pl.BlockSpec((BLK, D_IN), lambda i: (i, 0)),
                  pl.BlockSpec((D_IN, D_H), lambda i: (0, 0))],
        out_specs=pl.BlockSpec((BLK, D_H), lambda i: (i, 0)),
        out_shape=jax.ShapeDtypeStruct((N, D_H), jnp.float32),
    )(x, w1)


def _tc_scale(degs, xw):
    return pl.pallas_call(
        _scale_body,
        grid=(NBLK,),
        in_specs=[_DEG_SPEC,
                  pl.BlockSpec((BLK, D_H), lambda i: (i, 0))],
        out_specs=pl.BlockSpec((BLK, D_H), lambda i: (i, 0)),
        out_shape=jax.ShapeDtypeStruct((N, D_H), jnp.float32),
    )(degs, xw)


def _tc_mm2(degs, p1, b1):
    return pl.pallas_call(
        _mm2_body,
        grid=(NBLK,),
        in_specs=[_DEG_SPEC,
                  pl.BlockSpec((NC, BLK, D_H), lambda i: (0, i, 0)),
                  pl.BlockSpec((1, D_H), lambda i: (0, 0))],
        out_specs=pl.BlockSpec((BLK, D_H), lambda i: (i, 0)),
        out_shape=jax.ShapeDtypeStruct((N, D_H), jnp.float32),
    )(degs, p1, b1)


def _tc_out(degs, p2, w2, b2):
    return pl.pallas_call(
        _out_body,
        grid=(NBLK,),
        in_specs=[_DEG_SPEC,
                  pl.BlockSpec((NC, BLK, D_H), lambda i: (0, i, 0)),
                  pl.BlockSpec((D_H, C), lambda i: (0, 0)),
                  pl.BlockSpec((1, C), lambda i: (0, 0))],
        out_specs=pl.BlockSpec((BLK, C), lambda i: (i, 0)),
        out_shape=jax.ShapeDtypeStruct((N, C), jnp.float32),
    )(degs, p2, w2, b2)


def kernel(x, edge_index, W1, b1, W2, b2):
    src = edge_index[0]
    dst = edge_index[1]
    src2 = src.reshape(NW, NCHUNK_D, KD)
    dst2 = dst.reshape(NW, NCHUNK_D, KD)
    pad = EPT_P - EPT
    srcf = jnp.pad(src.reshape(NW, EPT), ((0, 0), (0, pad)))
    dstp = jnp.pad(dst.reshape(NW, EPT), ((0, 0), (0, pad)),
                   constant_values=N)
    ones1 = jnp.ones((KD,), jnp.float32)
    zeros1 = jnp.zeros((NP,), jnp.float32)
    zeros128 = jnp.zeros((NP, D_H), jnp.float32)

    xw = _tc_mm1(x, W1)
    degs = _sc_degrees(src2, dst2, ones1, zeros1)
    degs = degs.reshape(NC, 2, NP, 1)
    h1 = _tc_scale(degs, xw)
    p1 = _sc_aggregate(h1, srcf, dstp, zeros128)
    h2 = _tc_mm2(degs, p1, b1.reshape(1, D_H))
    p2 = _sc_aggregate(h2, srcf, dstp, zeros128)
    return _tc_out(degs, p2, W2, b2.reshape(1, C))


# speedup vs baseline: 1.4476x; 1.4476x over previous
import functools

import jax
import jax.numpy as jnp
from jax import lax
from jax.experimental import pallas as pl
from jax.experimental.pallas import tpu as pltpu
from jax.experimental.pallas import tpu_sc as plsc

N = 10000
E = 320000
D_IN = 128
D_H = 128
C = 64

NC = 2
NS = 16
NW = NC * NS
EPT = E // NW
KD = 40
NCHUNK_D = EPT // KD
KA = 80
NCHUNK_A = EPT // KA
NP = 10240
ROWS_PT = NP // NS

_SC_MESH = plsc.VectorSubcoreMesh(core_axis_name="c", subcore_axis_name="s",
                                  num_cores=NC, num_subcores=NS)


@functools.partial(
    pl.kernel,
    out_type=jax.ShapeDtypeStruct((NC, 2, NP), jnp.float32),
    mesh=_SC_MESH,
    scratch_types=[
        pltpu.VMEM((NCHUNK_D, KD), jnp.int32),
        pltpu.VMEM((NCHUNK_D, KD), jnp.int32),
        pltpu.VMEM((KD,), jnp.float32),
        pltpu.SemaphoreType.DMA,
        pltpu.SemaphoreType.DMA,
        pltpu.VMEM_SHARED((NP,), jnp.float32),
        pltpu.VMEM_SHARED((NP,), jnp.float32),
    ],
)
def _sc_degrees(src_hbm, dst_hbm, ones_hbm, zeros_hbm, out_hbm,
                sidx, didx, ones_v, sem_s, sem_d, acc_s, acc_d):
    c = lax.axis_index("c")
    s = lax.axis_index("s")
    wid = c * NS + s
    r0 = s * ROWS_PT
    pltpu.sync_copy(zeros_hbm.at[pl.ds(r0, ROWS_PT)], acc_s.at[pl.ds(r0, ROWS_PT)])
    pltpu.sync_copy(zeros_hbm.at[pl.ds(r0, ROWS_PT)], acc_d.at[pl.ds(r0, ROWS_PT)])
    pltpu.sync_copy(ones_hbm, ones_v)
    pltpu.sync_copy(src_hbm.at[wid], sidx)
    pltpu.sync_copy(dst_hbm.at[wid], didx)
    plsc.subcore_barrier()

    grp = 10

    def body(g, carry):
        c0 = g * grp
        for j in range(grp):
            pltpu.async_copy(ones_v, acc_s.at[sidx.at[c0 + j]], sem_s, add=True)
            pltpu.async_copy(ones_v, acc_d.at[didx.at[c0 + j]], sem_d, add=True)
        for j in range(grp):
            pltpu.make_async_copy(ones_v, acc_s.at[sidx.at[c0 + j]], sem_s).wait()
            pltpu.make_async_copy(ones_v, acc_d.at[didx.at[c0 + j]], sem_d).wait()
        return carry

    lax.fori_loop(0, NCHUNK_D // grp, body, 0)
    plsc.subcore_barrier()
    pltpu.sync_copy(acc_s.at[pl.ds(r0, ROWS_PT)], out_hbm.at[c, 0, pl.ds(r0, ROWS_PT)])
    pltpu.sync_copy(acc_d.at[pl.ds(r0, ROWS_PT)], out_hbm.at[c, 1, pl.ds(r0, ROWS_PT)])


@functools.partial(
    pl.kernel,
    out_type=jax.ShapeDtypeStruct((NC, NP, D_H), jnp.float32),
    mesh=_SC_MESH,
    scratch_types=[
        pltpu.VMEM((KA,), jnp.int32),
        pltpu.VMEM((KA,), jnp.int32),
        pltpu.VMEM((KA, D_H), jnp.float32),
        pltpu.SemaphoreType.DMA,
        pltpu.VMEM_SHARED((NP, D_H), jnp.float32),
    ],
)
def _sc_aggregate(h_hbm, src_hbm, dst_hbm, zeros_hbm, out_hbm,
                  sidx, didx, rows, sem, acc):
    c = lax.axis_index("c")
    s = lax.axis_index("s")
    wid = c * NS + s
    r0 = s * ROWS_PT
    pltpu.sync_copy(zeros_hbm.at[pl.ds(r0, ROWS_PT)], acc.at[pl.ds(r0, ROWS_PT)])
    plsc.subcore_barrier()

    base = wid * EPT

    def body(i, carry):
        off = base + i * KA
        pltpu.sync_copy(src_hbm.at[pl.ds(off, KA)], sidx)
        pltpu.sync_copy(dst_hbm.at[pl.ds(off, KA)], didx)
        pltpu.async_copy(h_hbm.at[sidx], rows, sem).wait()
        pltpu.sync_copy(rows, acc.at[didx], add=True)
        return carry

    lax.fori_loop(0, NCHUNK_A, body, 0)
    plsc.subcore_barrier()
    pltpu.sync_copy(acc.at[pl.ds(r0, ROWS_PT)], out_hbm.at[c, pl.ds(r0, ROWS_PT)])


BLK = 1000
NBLK = N // BLK


def _norm_cols(d):
    return jnp.where(d > 0, lax.rsqrt(d), 0.0)


def _mm1_body(x_ref, w_ref, o_ref):
    o_ref[...] = jnp.dot(x_ref[...], w_ref[...],
                         preferred_element_type=jnp.float32)


def _scale_body(degs_ref, xw_ref, o_ref):
    ns = _norm_cols(degs_ref[0, 0] + degs_ref[1, 0])
    o_ref[...] = xw_ref[...] * ns


def _mm2_body(degs_ref, p_ref, b1_ref, o_ref):
    ns = _norm_cols(degs_ref[0, 0] + degs_ref[1, 0])
    nd = _norm_cols(degs_ref[0, 1] + degs_ref[1, 1])
    a = p_ref[0] + p_ref[1]
    h = jnp.maximum(a * nd + b1_ref[...], 0.0)
    o_ref[...] = h * ns


def _out_body(degs_ref, p_ref, w_ref, b2_ref, o_ref):
    nd = _norm_cols(degs_ref[0, 1] + degs_ref[1, 1])
    a = (p_ref[0] + p_ref[1]) * nd
    o = jnp.dot(a, w_ref[...], preferred_element_type=jnp.float32) + b2_ref[...]
    m = jnp.max(o, axis=1, keepdims=True)
    e = jnp.exp(o - m)
    o_ref[...] = e / jnp.sum(e, axis=1, keepdims=True)


_DEG_SPEC = pl.BlockSpec((NC, 2, BLK, 1), lambda i: (0, 0, i, 0))


def _tc_mm1(x, w1):
    return pl.pallas_call(
        _mm1_body,
        grid=(NBLK,),
        in_specs=[pl.BlockSpec((BLK, D_IN), lambda i: (i, 0)),
                  pl.BlockSpec((D_IN, D_H), lambda i: (0, 0))],
        out_specs=pl.BlockSpec((BLK, D_H), lambda i: (i, 0)),
        out_shape=jax.ShapeDtypeStruct((N, D_H), jnp.float32),
    )(x, w1)


def _tc_scale(degs, xw):
    return pl.pallas_call(
        _scale_body,
        grid=(NBLK,),
        in_specs=[_DEG_SPEC,
                  pl.BlockSpec((BLK, D_H), lambda i: (i, 0))],
        out_specs=pl.BlockSpec((BLK, D_H), lambda i: (i, 0)),
        out_shape=jax.ShapeDtypeStruct((N, D_H), jnp.float32),
    )(degs, xw)


def _tc_mm2(degs, p1, b1):
    return pl.pallas_call(
        _mm2_body,
        grid=(NBLK,),
        in_specs=[_DEG_SPEC,
                  pl.BlockSpec((NC, BLK, D_H), lambda i: (0, i, 0)),
                  pl.BlockSpec((1, D_H), lambda i: (0, 0))],
        out_specs=pl.BlockSpec((BLK, D_H), lambda i: (i, 0)),
        out_shape=jax.ShapeDtypeStruct((N, D_H), jnp.float32),
    )(degs, p1, b1)


def _tc_out(degs, p2, w2, b2):
    return pl.pallas_call(
        _out_body,
        grid=(NBLK,),
        in_specs=[_DEG_SPEC,
                  pl.BlockSpec((NC, BLK, D_H), lambda i: (0, i, 0)),
                  pl.BlockSpec((D_H, C), lambda i: (0, 0)),
                  pl.BlockSpec((1, C), lambda i: (0, 0))],
        out_specs=pl.BlockSpec((BLK, C), lambda i: (i, 0)),
        out_shape=jax.ShapeDtypeStruct((N, C), jnp.float32),
    )(degs, p2, w2, b2)


def kernel(x, edge_index, W1, b1, W2, b2):
    src = edge_index[0]
    dst = edge_index[1]
    src2 = src.reshape(NW, NCHUNK_D, KD)
    dst2 = dst.reshape(NW, NCHUNK_D, KD)

    ones1 = jnp.ones((KD,), jnp.float32)
    zeros1 = jnp.zeros((NP,), jnp.float32)
    zeros128 = jnp.zeros((NP, D_H), jnp.float32)

    xw = _tc_mm1(x, W1)
    degs = _sc_degrees(src2, dst2, ones1, zeros1)
    degs = degs.reshape(NC, 2, NP, 1)
    h1 = _tc_scale(degs, xw)
    p1 = _sc_aggregate(h1, src, dst, zeros128)
    h2 = _tc_mm2(degs, p1, b1.reshape(1, D_H))
    p2 = _sc_aggregate(h2, src, dst, zeros128)
    return _tc_out(degs, p2, W2, b2.reshape(1, C))
